# Initial kernel scaffold; baseline (speedup 1.0000x reference)
import jax
import jax.numpy as jnp
from jax.experimental import pallas as pl


def kernel(X, edge_index, edge_weight, skip, H, C, p0, p1, lnp, fcp):
    raise NotImplementedError("write your pallas kernel here")


# speedup vs baseline: 1.0595x; 1.0595x over previous
import functools

import jax
import jax.numpy as jnp
from jax import lax
from jax.experimental import pallas as pl
from jax.experimental.pallas import tpu as pltpu

N = 50000
HID = 64
BR = 2000
f32 = jnp.float32


def _dot(a, b):
    return lax.dot_general(a, b, (((1,), (0,)), ((), ())),
                           precision=lax.Precision.HIGHEST,
                           preferred_element_type=f32)


def _ln(v, g, b):
    m = jnp.mean(v, axis=1, keepdims=True)
    var = jnp.mean((v - m) ** 2, axis=1, keepdims=True)
    return (v - m) * lax.rsqrt(var + 1e-5) * g + b


def _cell_body(x_r, h_r, c_r, ax_r, ah_r, Wx_r, Rx_r, Wh_r, Rh_r, b_r,
               wci_r, wcf_r, wco_r, gh_r, bh_r, gc_r, bc_r, go_r, bo_r,
               hln_o, cln_o, oln_o):
    pre = (_dot(ax_r[...], Wx_r[...]) + _dot(x_r[...], Rx_r[...])
           + _dot(ah_r[...], Wh_r[...]) + _dot(h_r[...], Rh_r[...]) + b_r[...])
    cc = c_r[...]
    gi = jax.nn.sigmoid(pre[:, 0:64] + wci_r[...] * cc)
    gf = jax.nn.sigmoid(pre[:, 64:128] + wcf_r[...] * cc)
    gt = jnp.tanh(pre[:, 128:192])
    cn = gf * cc + gi * gt
    go_gate = jax.nn.sigmoid(pre[:, 192:256] + wco_r[...] * cn)
    hn = go_gate * jnp.tanh(cn)
    hln_o[...] = _ln(hn, gh_r[...], bh_r[...])
    cln_o[...] = _ln(cn, gc_r[...], bc_r[...])
    oln_o[...] = jnp.maximum(_ln(hn, go_r[...], bo_r[...]), 0.0)


def _cell_tc(x, h, c, aggx, aggh, Wx, Rx, Wh, Rh, bias,
             wci, wcf, wco, gh, bh, gc, bc, go, bo):
    d = x.shape[1]
    row = lambda w: pl.BlockSpec((BR, w), lambda i: (i, 0))
    full = lambda a: pl.BlockSpec(a.shape, lambda i: (0,) * a.ndim)
    outs = [jax.ShapeDtypeStruct((N, HID), f32)] * 3
    return pl.pallas_call(
        _cell_body,
        grid=(N // BR,),
        in_specs=[row(d), row(HID), row(HID), row(d), row(HID)]
                 + [full(a) for a in (Wx, Rx, Wh, Rh, bias,
                                      wci, wcf, wco, gh, bh, gc, bc, go, bo)],
        out_specs=[row(HID)] * 3,
        out_shape=outs,
    )(x, h, c, aggx, aggh, Wx, Rx, Wh, Rh, bias,
      wci, wcf, wco, gh, bh, gc, bc, go, bo)


def _fc1_body(o_r, sk_r, ao_r, ask_r, W1a, W1b, R1a, R1b, b1, o2_o):
    pre = (_dot(ao_r[...], W1a[...]) + _dot(ask_r[...], W1b[...])
           + _dot(o_r[...], R1a[...]) + _dot(sk_r[...], R1b[...]) + b1[...])
    o2_o[...] = jnp.maximum(pre, 0.0)


def _fc1_tc(o, sk, ao, ask, W1a, W1b, R1a, R1b, b1):
    row = lambda w: pl.BlockSpec((BR, w), lambda i: (i, 0))
    full = lambda a: pl.BlockSpec(a.shape, lambda i: (0,) * a.ndim)
    return pl.pallas_call(
        _fc1_body,
        grid=(N // BR,),
        in_specs=[row(HID), row(2), row(HID), row(2)]
                 + [full(a) for a in (W1a, W1b, R1a, R1b, b1)],
        out_specs=row(HID),
        out_shape=jax.ShapeDtypeStruct((N, HID), f32),
    )(o, sk, ao, ask, W1a, W1b, R1a, R1b, b1)


def _fc2_body(ao2_r, o2_r, x0_r, W2, R2, b2, out_o):
    out_o[...] = (_dot(ao2_r[...], W2[...]) + _dot(o2_r[...], R2[...])
                  + b2[...] + x0_r[...])


def _fc2_tc(ao2, o2, x0, W2, R2, b2):
    row = lambda w: pl.BlockSpec((BR, w), lambda i: (i, 0))
    full = lambda a: pl.BlockSpec(a.shape, lambda i: (0,) * a.ndim)
    return pl.pallas_call(
        _fc2_body,
        grid=(N // BR,),
        in_specs=[row(HID), row(HID), row(1)]
                 + [full(a) for a in (W2, R2, b2)],
        out_specs=row(1),
        out_shape=jax.ShapeDtypeStruct((N, 1), f32),
    )(ao2, o2, x0, W2, R2, b2)


def _gate_stack(p, kx, kh):
    Wx = jnp.concatenate([p[kx + g] for g in 'ifco'], axis=1)
    Wh = jnp.concatenate([p[kh + g] for g in 'ifco'], axis=1)
    return Wx, Wh


def kernel(X, edge_index, edge_weight, skip, H, C, p0, p1, lnp, fcp):
    src = edge_index[0]
    dst = edge_index[1]
    ew = edge_weight

    def seg(x):
        return jax.ops.segment_sum(x[src] * ew[:, None], dst, num_segments=N)

    X8 = jnp.pad(jnp.concatenate([X, skip], axis=1), ((0, 0), (0, 3)))
    aggX8 = seg(X8)

    r1 = lambda a: a.reshape(1, -1)

    def run_cell(p, d, x, hh, cc, ax, ah):
        Wx, Rx = _gate_stack(p, 'Wx', 'Rx')
        Wh, Rh = _gate_stack(p, 'Wh', 'Rh')
        if d != Wx.shape[0]:
            Wx = jnp.pad(Wx, ((0, d - Wx.shape[0]), (0, 0)))
            Rx = jnp.pad(Rx, ((0, d - Rx.shape[0]), (0, 0)))
        bias = jnp.concatenate([p['b' + g] for g in 'ifco']).reshape(1, 256)
        return _cell_tc(x, hh, cc, ax, ah, Wx, Rx, Wh, Rh, bias,
                        r1(p['wci']), r1(p['wcf']), r1(p['wco']),
                        r1(lnp['gh']), r1(lnp['bh']), r1(lnp['gc']),
                        r1(lnp['bc']), r1(lnp['go']), r1(lnp['bo']))

    h0ln, c0ln, _ = run_cell(p0, 8, X8, H[0], C[0], aggX8, seg(H[0]))
    h1ln, c1ln, oln = run_cell(p1, 64, h0ln, H[1], C[1], seg(h0ln), seg(H[1]))

    aggo = seg(oln)
    aggskip = aggX8[:, 3:5]
    W1, R1 = fcp['W1'], fcp['R1']
    o2 = _fc1_tc(oln, skip, aggo, aggskip,
                 W1[:HID], W1[HID:], R1[:HID], R1[HID:], r1(fcp['b1']))
    res = _fc2_tc(seg(o2), o2, X[:, 0:1], fcp['W2'], fcp['R2'], r1(fcp['b2']))
    hidden = jnp.stack([h0ln, h1ln])
    cell = jnp.stack([c0ln, c1ln])
    return (res, hidden, cell)
